# Initial kernel scaffold; baseline (speedup 1.0000x reference)
import jax
import jax.numpy as jnp
from jax.experimental import pallas as pl


def kernel(inputs, kernel):
    raise NotImplementedError("write your pallas kernel here")


# speedup vs baseline: 5.7355x; 5.7355x over previous
import functools

import jax
import jax.numpy as jnp
from jax import lax
from jax.experimental import pallas as pl
from jax.experimental.pallas import tpu as pltpu
from jax.experimental.pallas import tpu_sc as plsc

_NC = 2
_NS = 16
_NW = _NC * _NS

_IDX_PER_STREAM = 128
_STREAMS_PER_CHUNK = 8
_CHUNK = _IDX_PER_STREAM * _STREAMS_PER_CHUNK


def _transpose_tc(w, vocab_pad, block_w):
    units = w.shape[0]

    def body(in_ref, out_ref):
        out_ref[...] = in_ref[...].T

    return pl.pallas_call(
        body,
        grid=(vocab_pad // block_w,),
        in_specs=[pl.BlockSpec((units, block_w), lambda i: (0, i))],
        out_specs=pl.BlockSpec((block_w, units), lambda i: (i, 0)),
        out_shape=jax.ShapeDtypeStruct((vocab_pad, units), w.dtype),
    )(w)


def _make_gather(vocab_pad, units, b):
    b_per_w = b // _NW
    n_chunks = b_per_w // _CHUNK
    chunk_rows = _CHUNK // _IDX_PER_STREAM

    mesh = plsc.VectorSubcoreMesh(core_axis_name="c", subcore_axis_name="s")

    @functools.partial(
        pl.kernel,
        mesh=mesh,
        compiler_params=pltpu.CompilerParams(use_tc_tiling_on_sc=False),
        out_type=jax.ShapeDtypeStruct((b, units), jnp.float32),
        scratch_types=[
            pltpu.VMEM((chunk_rows, _IDX_PER_STREAM), jnp.int32),
            pltpu.VMEM((_CHUNK, units), jnp.float32),
            pltpu.SemaphoreType.DMA,
        ],
    )
    def gather_kernel(table_hbm, idx_hbm, out_hbm, idx_v, rows_v, gat_sem):
        wid = lax.axis_index("s") * _NC + lax.axis_index("c")
        base_row = wid * (b_per_w // _IDX_PER_STREAM)

        def body(g, _):
            pltpu.sync_copy(
                idx_hbm.at[pl.ds(base_row + g * chunk_rows, chunk_rows)],
                idx_v)
            for j in range(_STREAMS_PER_CHUNK):
                pltpu.async_copy(
                    table_hbm.at[idx_v.at[j]],
                    rows_v.at[pl.ds(j * _IDX_PER_STREAM, _IDX_PER_STREAM)],
                    gat_sem)
            for j in range(_STREAMS_PER_CHUNK):
                pltpu.make_async_copy(
                    table_hbm.at[idx_v.at[j]],
                    rows_v.at[pl.ds(j * _IDX_PER_STREAM, _IDX_PER_STREAM)],
                    gat_sem).wait()
            pltpu.sync_copy(
                rows_v,
                out_hbm.at[pl.ds(wid * b_per_w + g * _CHUNK, _CHUNK)])
            return ()

        lax.fori_loop(0, n_chunks, body, (), unroll=False)

    return gather_kernel


def kernel(inputs, kernel):
    units, vocab = kernel.shape
    batch, hist = inputs.shape
    b = batch * hist

    vocab_pad = 102400
    w = jnp.pad(kernel, ((0, 0), (0, vocab_pad - vocab)))
    table = _transpose_tc(w, vocab_pad, block_w=4096)

    idx = inputs.astype(jnp.int32).reshape(b // _IDX_PER_STREAM,
                                           _IDX_PER_STREAM)
    out = _make_gather(vocab_pad, units, b)(table, idx)
    return out.reshape(batch, hist, units)
